# baseline (speedup 1.0000x reference)
import functools
import math

import jax
import jax.numpy as jnp
from jax import lax
from jax.experimental import pallas as pl
from jax.experimental.pallas import tpu as pltpu
from jax.experimental.pallas import tpu_sc as plsc

N = 10000
E = 320000
D_FEAT = 128
HID = 32
ZW = 128
NC, NS = 2, 16
NW = NC * NS
EPT = E // NW
CB = 112
NCH = 91
PAD = NCH * CB - EPT
ACCN = N + 16
SLAB = 624
BN_SCALE = 1.0 / math.sqrt(1.0 + 1e-5)
NB = 10
BLK = N // NB


def _k1_body(x_ref, wz_ref, al_ref, wp_ref, ar_ref,
             ztab_ref, proj_ref, ers_ref):
    x = x_ref[...]
    z = jnp.dot(x, wz_ref[...], preferred_element_type=jnp.float32)
    els = lax.dot_general(
        z, al_ref[...], (((1,), (1,)), ((), ())),
        preferred_element_type=jnp.float32)
    ones2 = jnp.ones((x.shape[0], 2), jnp.float32)
    pad60 = jnp.zeros((x.shape[0], 60), jnp.float32)
    ztab_ref[...] = jnp.concatenate([z, ones2, els, pad60], axis=1)
    proj_ref[...] = jnp.dot(x, wp_ref[...], preferred_element_type=jnp.float32)
    ers_ref[...] = lax.dot_general(
        z, ar_ref[...], (((1,), (1,)), ((), ())),
        preferred_element_type=jnp.float32)


def _k1(x, wz, al, wp, ar):
    return pl.pallas_call(
        _k1_body,
        grid=(NB,),
        in_specs=[
            pl.BlockSpec((BLK, D_FEAT), lambda i: (i, 0)),
            pl.BlockSpec((D_FEAT, 2 * HID), lambda i: (0, 0)),
            pl.BlockSpec((2, 2 * HID), lambda i: (0, 0)),
            pl.BlockSpec((D_FEAT, HID), lambda i: (0, 0)),
            pl.BlockSpec((2, 2 * HID), lambda i: (0, 0)),
        ],
        out_specs=[
            pl.BlockSpec((BLK, ZW), lambda i: (i, 0)),
            pl.BlockSpec((BLK, HID), lambda i: (i, 0)),
            pl.BlockSpec((BLK, 2), lambda i: (i, 0)),
        ],
        out_shape=[
            jax.ShapeDtypeStruct((N, ZW), jnp.float32),
            jax.ShapeDtypeStruct((N, HID), jnp.float32),
            jax.ShapeDtypeStruct((N, 2), jnp.float32),
        ],
    )(x, wz, al, wp, ar)


def _sc_body(ztab, er0, er1, edm, zrs, out,
             eds_a, eds_b, src_a, dst_a, src_b, dst_b,
             er0_v, er1_v, zr_a, zr_b, acc, sem_a, sem_b):
    c = lax.axis_index("c")
    s = lax.axis_index("s")
    wid = s * NC + c
    base = s * SLAB
    pltpu.sync_copy(zrs, acc.at[pl.ds(base, SLAB)])

    @pl.when(s == NS - 1)
    def _zero_tail():
        pltpu.sync_copy(zrs.at[pl.ds(0, ACCN - NS * SLAB)],
                        acc.at[pl.ds(NS * SLAB, ACCN - NS * SLAB)])

    pltpu.sync_copy(er0, er0_v.at[pl.ds(0, N)])
    pltpu.sync_copy(er1, er1_v.at[pl.ds(0, N)])
    plsc.subcore_barrier()

    iota = lax.iota(jnp.int32, 16)
    c66 = jnp.full((16,), 66, jnp.int32)
    c67 = jnp.full((16,), 67, jnp.int32)

    def stage(j, eds, srcb, dstb):
        pltpu.sync_copy(edm.at[wid, j], eds)
        for g in range(CB // 16):
            gs = pl.ds(g * 16, 16)
            srcb[gs] = eds[0, gs].astype(jnp.int32)
            dstb[gs] = eds[1, gs].astype(jnp.int32)

    def compute_scatter(zr, eds, dstb):
        for g in range(CB // 16):
            eidx = iota + g * 16
            dstv = dstb[pl.ds(g * 16, 16)]
            ewv = eds[2, pl.ds(g * 16, 16)]
            el0v = plsc.load_gather(zr, [eidx, c66])
            el1v = plsc.load_gather(zr, [eidx, c67])
            er0v = plsc.load_gather(er0_v, [dstv])
            er1v = plsc.load_gather(er1_v, [dstv])
            t0 = (el0v + er0v) * ewv
            t1 = (el1v + er1v) * ewv
            p0 = jnp.exp(jnp.maximum(t0, t0 * 0.01))
            p1 = jnp.exp(jnp.maximum(t1, t1 * 0.01))
            for i in range(16):
                e = g * 16 + i
                isp = jnp.full((16,), i, jnp.int32)
                p0s = jnp.take(p0, isp)
                p1s = jnp.take(p1, isp)
                zr[e, pl.ds(0, 16)] = zr[e, pl.ds(0, 16)] * p0s
                zr[e, pl.ds(16, 16)] = zr[e, pl.ds(16, 16)] * p0s
                zr[e, pl.ds(32, 16)] = zr[e, pl.ds(32, 16)] * p1s
                zr[e, pl.ds(48, 16)] = zr[e, pl.ds(48, 16)] * p1s
                zr[e, pl.ds(64, 16)] = jnp.where(
                    iota == 0, p0s, jnp.where(iota == 1, p1s, 0.0))
        pltpu.sync_copy(zr, acc.at[dstb], add=True)

    stage(0, eds_a, src_a, dst_a)
    pltpu.async_copy(ztab.at[src_a], zr_a, sem_a)

    def body(i, carry):
        j = 2 * i
        stage(j + 1, eds_b, src_b, dst_b)
        pltpu.async_copy(ztab.at[src_b], zr_b, sem_b)
        pltpu.make_async_copy(ztab.at[src_a], zr_a, sem_a).wait()
        compute_scatter(zr_a, eds_a, dst_a)
        stage(j + 2, eds_a, src_a, dst_a)
        pltpu.async_copy(ztab.at[src_a], zr_a, sem_a)
        pltpu.make_async_copy(ztab.at[src_b], zr_b, sem_b).wait()
        compute_scatter(zr_b, eds_b, dst_b)
        return carry

    lax.fori_loop(0, (NCH - 1) // 2, body, 0)
    pltpu.make_async_copy(ztab.at[src_a], zr_a, sem_a).wait()
    compute_scatter(zr_a, eds_a, dst_a)
    plsc.subcore_barrier()
    pltpu.sync_copy(acc.at[pl.ds(base, SLAB)], out.at[c, pl.ds(base, SLAB)])

    @pl.when(s == NS - 1)
    def _out_tail():
        pltpu.sync_copy(acc.at[pl.ds(NS * SLAB, 16)],
                        out.at[c, pl.ds(NS * SLAB, 16)])


def _sc(ztab, er0, er1, edm, zrs):
    mesh = plsc.VectorSubcoreMesh(core_axis_name="c", subcore_axis_name="s")
    f = functools.partial(
        pl.kernel,
        out_type=jax.ShapeDtypeStruct((2, N, ZW), jnp.float32),
        mesh=mesh,
        compiler_params=pltpu.CompilerParams(needs_layout_passes=False),
        scratch_types=[
            pltpu.VMEM((3, CB), jnp.float32),
            pltpu.VMEM((3, CB), jnp.float32),
            pltpu.VMEM((CB,), jnp.int32),
            pltpu.VMEM((CB,), jnp.int32),
            pltpu.VMEM((CB,), jnp.int32),
            pltpu.VMEM((CB,), jnp.int32),
            pltpu.VMEM((N + 16,), jnp.float32),
            pltpu.VMEM((N + 16,), jnp.float32),
            pltpu.VMEM((CB, ZW), jnp.float32),
            pltpu.VMEM((CB, ZW), jnp.float32),
            pltpu.VMEM_SHARED((ACCN, ZW), jnp.float32),
            pltpu.SemaphoreType.DMA,
            pltpu.SemaphoreType.DMA,
        ],
    )(_sc_body)
    return f(ztab, er0, er1, edm, zrs)


def _k3_body(a0_ref, a1_ref, pr_ref, g_ref, b_ref, o_ref):
    acc = a0_ref[...] + a1_ref[...]
    d0 = acc[:, 64:65] + 1e-16
    d1 = acc[:, 65:66] + 1e-16
    hg = 0.5 * (acc[:, 0:32] / d0 + acc[:, 32:64] / d1)
    hcat = jnp.concatenate([pr_ref[...], hg], axis=1)
    o_ref[...] = g_ref[...] * (hcat * BN_SCALE) + b_ref[...]


def _k3(a0, a1, proj, gamma, beta):
    return pl.pallas_call(
        _k3_body,
        grid=(NB,),
        in_specs=[
            pl.BlockSpec((BLK, ZW), lambda i: (i, 0)),
            pl.BlockSpec((BLK, ZW), lambda i: (i, 0)),
            pl.BlockSpec((BLK, HID), lambda i: (i, 0)),
            pl.BlockSpec((1, 2 * HID), lambda i: (0, 0)),
            pl.BlockSpec((1, 2 * HID), lambda i: (0, 0)),
        ],
        out_specs=pl.BlockSpec((BLK, 2 * HID), lambda i: (i, 0)),
        out_shape=jax.ShapeDtypeStruct((N, 2 * HID), jnp.float32),
    )(a0, a1, proj, gamma, beta)


def kernel(x, edge_index, edge_w, W_proj, Wa, att_l, att_r, bn_gamma, bn_beta):
    srcw = edge_index[0].astype(jnp.float32).reshape(NW, EPT)
    dstw = edge_index[1].astype(jnp.float32).reshape(NW, EPT)
    eww = edge_w.reshape(NW, EPT)
    srcp = jnp.concatenate([srcw, jnp.zeros((NW, PAD), jnp.float32)], axis=1)
    dstp = jnp.concatenate(
        [dstw, jnp.full((NW, PAD), float(N), jnp.float32)], axis=1)
    ewp = jnp.concatenate([eww, jnp.zeros((NW, PAD), jnp.float32)], axis=1)
    edm = jnp.stack([srcp, dstp, ewp], axis=1)
    edm = edm.reshape(NW, 3, NCH, CB).transpose(0, 2, 1, 3)
    wz = jnp.concatenate([Wa[0], Wa[1]], axis=1)
    zr = jnp.zeros((HID,), jnp.float32)
    al = jnp.stack([jnp.concatenate([att_l[0], zr]),
                    jnp.concatenate([zr, att_l[1]])], axis=0)
    ar = jnp.stack([jnp.concatenate([att_r[0], zr]),
                    jnp.concatenate([zr, att_r[1]])], axis=0)
    wp = W_proj.T

    ztab, proj, ers = _k1(x, wz, al, wp, ar)
    zrs = jnp.zeros((SLAB, ZW), jnp.float32)
    partials = _sc(ztab, ers[:, 0], ers[:, 1], edm, zrs)
    out = _k3(partials[0], partials[1], proj,
              bn_gamma.reshape(1, 2 * HID), bn_beta.reshape(1, 2 * HID))
    return out.reshape(1, N, 1, 2 * HID)


# speedup vs baseline: 1.3393x; 1.3393x over previous
import functools
import math

import jax
import jax.numpy as jnp
from jax import lax
from jax.experimental import pallas as pl
from jax.experimental.pallas import tpu as pltpu
from jax.experimental.pallas import tpu_sc as plsc

N = 10000
E = 320000
D_FEAT = 128
HID = 32
ZW = 128
NC, NS = 2, 16
NW = NC * NS
EPT = E // NW
CB = 80
NCH = EPT // CB
SLAB = 624
BN_SCALE = 1.0 / math.sqrt(1.0 + 1e-5)
NB = 10
BLK = N // NB


def _k1_body(x_ref, wz_ref, al_ref, wp_ref, ar_ref,
             ztab_ref, proj_ref, ers_ref):
    x = x_ref[...]
    z = jnp.dot(x, wz_ref[...], preferred_element_type=jnp.float32)
    els = lax.dot_general(
        z, al_ref[...], (((1,), (1,)), ((), ())),
        preferred_element_type=jnp.float32)
    ones2 = jnp.ones((x.shape[0], 2), jnp.float32)
    pad60 = jnp.zeros((x.shape[0], 60), jnp.float32)
    ztab_ref[...] = jnp.concatenate([z, ones2, els, pad60], axis=1)
    proj_ref[...] = jnp.dot(x, wp_ref[...], preferred_element_type=jnp.float32)
    ers_ref[...] = lax.dot_general(
        z, ar_ref[...], (((1,), (1,)), ((), ())),
        preferred_element_type=jnp.float32)


def _k1(x, wz, al, wp, ar):
    return pl.pallas_call(
        _k1_body,
        grid=(NB,),
        in_specs=[
            pl.BlockSpec((BLK, D_FEAT), lambda i: (i, 0)),
            pl.BlockSpec((D_FEAT, 2 * HID), lambda i: (0, 0)),
            pl.BlockSpec((2, 2 * HID), lambda i: (0, 0)),
            pl.BlockSpec((D_FEAT, HID), lambda i: (0, 0)),
            pl.BlockSpec((2, 2 * HID), lambda i: (0, 0)),
        ],
        out_specs=[
            pl.BlockSpec((BLK, ZW), lambda i: (i, 0)),
            pl.BlockSpec((BLK, HID), lambda i: (i, 0)),
            pl.BlockSpec((BLK, 2), lambda i: (i, 0)),
        ],
        out_shape=[
            jax.ShapeDtypeStruct((N, ZW), jnp.float32),
            jax.ShapeDtypeStruct((N, HID), jnp.float32),
            jax.ShapeDtypeStruct((N, 2), jnp.float32),
        ],
    )(x, wz, al, wp, ar)


def _sc_body(ztab, er0, er1, srcm, dstm, ewm, zrs, out,
             src_a, dst_a, ew_a, src_b, dst_b, ew_b,
             er0_v, er1_v, zr_a, zr_b, acc, sem_a, sem_b):
    c = lax.axis_index("c")
    s = lax.axis_index("s")
    wid = s * NC + c
    base = s * SLAB
    pltpu.sync_copy(zrs, acc.at[pl.ds(base, SLAB)])

    @pl.when(s == NS - 1)
    def _zero_tail():
        pltpu.sync_copy(zrs.at[pl.ds(0, 16)], acc.at[pl.ds(NS * SLAB, 16)])

    pltpu.sync_copy(er0, er0_v)
    pltpu.sync_copy(er1, er1_v)
    plsc.subcore_barrier()

    iota = lax.iota(jnp.int32, 16)
    c66 = jnp.full((16,), 66, jnp.int32)
    c67 = jnp.full((16,), 67, jnp.int32)

    def stage(j, srcb, dstb, ewb):
        pltpu.sync_copy(srcm.at[wid, j], srcb)
        pltpu.sync_copy(dstm.at[wid, j], dstb)
        pltpu.sync_copy(ewm.at[wid, j], ewb)

    def compute_scatter(zr, dstb, ewb):
        for g in range(CB // 16):
            eidx = iota + g * 16
            dstv = dstb[pl.ds(g * 16, 16)]
            ewv = ewb[pl.ds(g * 16, 16)]
            el0v = plsc.load_gather(zr, [eidx, c66])
            el1v = plsc.load_gather(zr, [eidx, c67])
            er0v = plsc.load_gather(er0_v, [dstv])
            er1v = plsc.load_gather(er1_v, [dstv])
            t0 = (el0v + er0v) * ewv
            t1 = (el1v + er1v) * ewv
            p0 = jnp.exp(jnp.maximum(t0, t0 * 0.01))
            p1 = jnp.exp(jnp.maximum(t1, t1 * 0.01))
            for i in range(16):
                e = g * 16 + i
                isp = jnp.full((16,), i, jnp.int32)
                p0s = jnp.take(p0, isp)
                p1s = jnp.take(p1, isp)
                zr[e, pl.ds(0, 16)] = zr[e, pl.ds(0, 16)] * p0s
                zr[e, pl.ds(16, 16)] = zr[e, pl.ds(16, 16)] * p0s
                zr[e, pl.ds(32, 16)] = zr[e, pl.ds(32, 16)] * p1s
                zr[e, pl.ds(48, 16)] = zr[e, pl.ds(48, 16)] * p1s
                zr[e, pl.ds(64, 16)] = jnp.where(
                    iota == 0, p0s, jnp.where(iota == 1, p1s, 0.0))
        pltpu.sync_copy(zr, acc.at[dstb], add=True)

    stage(0, src_a, dst_a, ew_a)
    pltpu.async_copy(ztab.at[src_a], zr_a, sem_a)

    def body(i, carry):
        j = 2 * i
        stage(j + 1, src_b, dst_b, ew_b)
        pltpu.async_copy(ztab.at[src_b], zr_b, sem_b)
        pltpu.make_async_copy(ztab.at[src_a], zr_a, sem_a).wait()
        compute_scatter(zr_a, dst_a, ew_a)
        stage(j + 2, src_a, dst_a, ew_a)
        pltpu.async_copy(ztab.at[src_a], zr_a, sem_a)
        pltpu.make_async_copy(ztab.at[src_b], zr_b, sem_b).wait()
        compute_scatter(zr_b, dst_b, ew_b)
        return carry

    lax.fori_loop(0, (NCH - 1) // 2, body, 0)
    pltpu.make_async_copy(ztab.at[src_a], zr_a, sem_a).wait()
    compute_scatter(zr_a, dst_a, ew_a)
    plsc.subcore_barrier()
    pltpu.sync_copy(acc.at[pl.ds(base, SLAB)], out.at[c, pl.ds(base, SLAB)])

    @pl.when(s == NS - 1)
    def _out_tail():
        pltpu.sync_copy(acc.at[pl.ds(NS * SLAB, 16)],
                        out.at[c, pl.ds(NS * SLAB, 16)])


def _sc(ztab, er0, er1, srcm, dstm, ewm, zrs):
    mesh = plsc.VectorSubcoreMesh(core_axis_name="c", subcore_axis_name="s")
    f = functools.partial(
        pl.kernel,
        out_type=jax.ShapeDtypeStruct((2, N, ZW), jnp.float32),
        mesh=mesh,
        compiler_params=pltpu.CompilerParams(needs_layout_passes=False),
        scratch_types=[
            pltpu.VMEM((CB,), jnp.int32),
            pltpu.VMEM((CB,), jnp.int32),
            pltpu.VMEM((CB,), jnp.float32),
            pltpu.VMEM((CB,), jnp.int32),
            pltpu.VMEM((CB,), jnp.int32),
            pltpu.VMEM((CB,), jnp.float32),
            pltpu.VMEM((N,), jnp.float32),
            pltpu.VMEM((N,), jnp.float32),
            pltpu.VMEM((CB, ZW), jnp.float32),
            pltpu.VMEM((CB, ZW), jnp.float32),
            pltpu.VMEM_SHARED((N, ZW), jnp.float32),
            pltpu.SemaphoreType.DMA,
            pltpu.SemaphoreType.DMA,
        ],
    )(_sc_body)
    return f(ztab, er0, er1, srcm, dstm, ewm, zrs)


def _k3_body(a0_ref, a1_ref, pr_ref, g_ref, b_ref, o_ref):
    acc = a0_ref[...] + a1_ref[...]
    d0 = acc[:, 64:65] + 1e-16
    d1 = acc[:, 65:66] + 1e-16
    hg = 0.5 * (acc[:, 0:32] / d0 + acc[:, 32:64] / d1)
    hcat = jnp.concatenate([pr_ref[...], hg], axis=1)
    o_ref[...] = g_ref[...] * (hcat * BN_SCALE) + b_ref[...]


def _k3(a0, a1, proj, gamma, beta):
    return pl.pallas_call(
        _k3_body,
        grid=(NB,),
        in_specs=[
            pl.BlockSpec((BLK, ZW), lambda i: (i, 0)),
            pl.BlockSpec((BLK, ZW), lambda i: (i, 0)),
            pl.BlockSpec((BLK, HID), lambda i: (i, 0)),
            pl.BlockSpec((1, 2 * HID), lambda i: (0, 0)),
            pl.BlockSpec((1, 2 * HID), lambda i: (0, 0)),
        ],
        out_specs=pl.BlockSpec((BLK, 2 * HID), lambda i: (i, 0)),
        out_shape=jax.ShapeDtypeStruct((N, 2 * HID), jnp.float32),
    )(a0, a1, proj, gamma, beta)


def kernel(x, edge_index, edge_w, W_proj, Wa, att_l, att_r, bn_gamma, bn_beta):
    src = edge_index[0].astype(jnp.int32).reshape(NW, NCH, CB)
    dst = edge_index[1].astype(jnp.int32).reshape(NW, NCH, CB)
    ew = edge_w.reshape(NW, NCH, CB)
    wz = jnp.concatenate([Wa[0], Wa[1]], axis=1)
    zr = jnp.zeros((HID,), jnp.float32)
    al = jnp.stack([jnp.concatenate([att_l[0], zr]),
                    jnp.concatenate([zr, att_l[1]])], axis=0)
    ar = jnp.stack([jnp.concatenate([att_r[0], zr]),
                    jnp.concatenate([zr, att_r[1]])], axis=0)
    wp = W_proj.T

    ztab, proj, ers = _k1(x, wz, al, wp, ar)
    zrs = jnp.zeros((SLAB, ZW), jnp.float32)
    partials = _sc(ztab, ers[:, 0], ers[:, 1], src, dst, ew, zrs)
    out = _k3(partials[0], partials[1], proj,
              bn_gamma.reshape(1, 2 * HID), bn_beta.reshape(1, 2 * HID))
    return out.reshape(1, N, 1, 2 * HID)
